# baseline (speedup 1.0000x reference)
import jax
import jax.numpy as jnp
from jax import lax
from jax.experimental import pallas as pl
from jax.experimental.pallas import tpu as pltpu
from jax.experimental.pallas import tpu_sc as plsc

VOCAB = 100000
D = 1024
B = 4 * 4096
NC = 2
NS = 16
NW = NC * NS
PER_W = B // NW
C = 32
NCHUNK = PER_W // C
NB = 3
LANES = 16
SCALE = 32.0
XCOLS = 4096
W_PER_XROW = XCOLS // PER_W


def _sc_body(idx_hbm, table_hbm, out_hbm, idx_v, buf, *sems):
    gsems = sems[:NB]
    osems = sems[NB:]
    wid = lax.axis_index("s") * NC + lax.axis_index("c")
    row0 = wid * PER_W

    pltpu.sync_copy(
        idx_hbm.at[wid // W_PER_XROW,
                   pl.ds((wid % W_PER_XROW) * PER_W, PER_W)], idx_v)

    def start_gather(g, s):
        pltpu.async_copy(table_hbm.at[idx_v.at[pl.ds(g * C, C)]],
                         buf.at[s], gsems[s])

    def wait_gather(g, s):
        pltpu.make_async_copy(table_hbm.at[idx_v.at[pl.ds(g * C, C)]],
                              buf.at[s], gsems[s]).wait()

    H = C // 2

    def start_out_half(g, s, h):
        pltpu.async_copy(buf.at[s, pl.ds(h * H, H)],
                         out_hbm.at[pl.ds(row0 + g * C + h * H, H)],
                         osems[2 * s + h])

    def wait_out_half(g, s, h):
        pltpu.make_async_copy(buf.at[s, pl.ds(h * H, H)],
                              out_hbm.at[pl.ds(row0 + g * C + h * H, H)],
                              osems[2 * s + h]).wait()

    def scale_half(s, h):
        @pl.loop(h * H, (h + 1) * H)
        def _(r):
            for c in range(D // LANES):
                sl = pl.ds(c * LANES, LANES)
                buf[s, r, sl] = buf[s, r, sl] * SCALE

    start_gather(0, 0)
    start_gather(1, 1)
    start_gather(2, 2)

    for g in range(NCHUNK):
        s = g % NB
        wait_gather(g, s)
        scale_half(s, 0)
        start_out_half(g, s, 0)
        if g >= 1 and g + 2 < NCHUNK:
            t = (g - 1) % NB
            wait_out_half(g - 1, t, 0)
            wait_out_half(g - 1, t, 1)
            start_gather(g + 2, (g + 2) % NB)
        scale_half(s, 1)
        start_out_half(g, s, 1)

    for g in range(NCHUNK - 3, NCHUNK):
        wait_out_half(g, g % NB, 0)
        wait_out_half(g, g % NB, 1)


def kernel(x, table):
    call = pl.kernel(
        _sc_body,
        out_type=jax.ShapeDtypeStruct((B, D), jnp.float32),
        mesh=plsc.VectorSubcoreMesh(
            core_axis_name="c", subcore_axis_name="s",
            num_cores=NC, num_subcores=NS),
        scratch_types=[
            pltpu.VMEM((PER_W,), jnp.int32),
            pltpu.VMEM((NB, C, D), jnp.float32),
        ] + [pltpu.SemaphoreType.DMA] * (3 * NB),
    )
    out = call(x.astype(jnp.int32), table)
    return out.reshape(x.shape + (D,))


# speedup vs baseline: 1.1438x; 1.1438x over previous
import jax
import jax.numpy as jnp
from jax import lax
from jax.experimental import pallas as pl
from jax.experimental.pallas import tpu as pltpu
from jax.experimental.pallas import tpu_sc as plsc

VOCAB = 100000
D = 1024
B = 4 * 4096
NC = 2
NS = 16
NW = NC * NS
PER_W = B // NW
C = 32
NCHUNK = PER_W // C
NB = 3
LANES = 16
SCALE = 32.0
XCOLS = 4096
W_PER_XROW = XCOLS // PER_W


def _sc_body(idx_hbm, table_hbm, out_hbm, idx_v, buf, *sems):
    gsems = sems[:NB]
    osems = sems[NB:]
    wid = lax.axis_index("s") * NC + lax.axis_index("c")
    row0 = wid * PER_W

    pltpu.sync_copy(
        idx_hbm.at[wid // W_PER_XROW,
                   pl.ds((wid % W_PER_XROW) * PER_W, PER_W)], idx_v)

    def start_gather(g, s):
        pltpu.async_copy(table_hbm.at[idx_v.at[pl.ds(g * C, C)]],
                         buf.at[s], gsems[s])

    def wait_gather(g, s):
        pltpu.make_async_copy(table_hbm.at[idx_v.at[pl.ds(g * C, C)]],
                              buf.at[s], gsems[s]).wait()

    def start_out(g, s):
        pltpu.async_copy(buf.at[s], out_hbm.at[pl.ds(row0 + g * C, C)],
                         osems[s])

    def wait_out(g, s):
        pltpu.make_async_copy(buf.at[s],
                              out_hbm.at[pl.ds(row0 + g * C, C)],
                              osems[s]).wait()

    def scale(s):
        @pl.loop(0, C)
        def _(r):
            for c in range(D // LANES):
                sl = pl.ds(c * LANES, LANES)
                buf[s, r, sl] = buf[s, r, sl] * SCALE

    start_gather(0, 0)
    start_gather(1, 1)
    start_gather(2, 2)

    for g in range(3):
        s = g % NB
        wait_gather(g, s)
        scale(s)
        start_out(g, s)
        if g >= 1:
            wait_out(g - 1, (g - 1) % NB)
            start_gather(g + 2, (g + 2) % NB)

    @pl.loop(1, NCHUNK // NB)
    def _(p):
        for s_off in range(NB):
            g = p * NB + s_off
            s = s_off % NB
            wait_gather(g, s)
            scale(s)
            start_out(g, s)
            wait_out(g - 1, (s_off - 1) % NB)
            @pl.when(g + 2 < NCHUNK)
            def _():
                start_gather(g + 2, (s_off + 2) % NB)

    g = NCHUNK - 1
    s = g % NB
    wait_gather(g, s)
    scale(s)
    start_out(g, s)

    for g in range(NCHUNK - 2, NCHUNK):
        wait_out(g, g % NB)


def kernel(x, table):
    call = pl.kernel(
        _sc_body,
        out_type=jax.ShapeDtypeStruct((B, D), jnp.float32),
        mesh=plsc.VectorSubcoreMesh(
            core_axis_name="c", subcore_axis_name="s",
            num_cores=NC, num_subcores=NS),
        scratch_types=[
            pltpu.VMEM((PER_W,), jnp.int32),
            pltpu.VMEM((NB, C, D), jnp.float32),
        ] + [pltpu.SemaphoreType.DMA] * (3 * NB),
    )
    out = call(x.astype(jnp.int32), table)
    return out.reshape(x.shape + (D,))


# speedup vs baseline: 1.1496x; 1.0050x over previous
import jax
import jax.numpy as jnp
from jax import lax
from jax.experimental import pallas as pl
from jax.experimental.pallas import tpu as pltpu
from jax.experimental.pallas import tpu_sc as plsc

VOCAB = 100000
D = 1024
B = 4 * 4096
NC = 2
NS = 16
NW = NC * NS
PER_W = B // NW
C = 32
NCHUNK = PER_W // C
NB = 3
LANES = 16
SCALE = 32.0
XCOLS = 4096
W_PER_XROW = XCOLS // PER_W


def _sc_body(idx_hbm, table_hbm, out_hbm, idx_v, buf, *sems):
    gsems = sems[:NB]
    osems = sems[NB:]
    wid = lax.axis_index("s") * NC + lax.axis_index("c")
    row0 = wid * PER_W

    pltpu.sync_copy(
        idx_hbm.at[wid // W_PER_XROW,
                   pl.ds((wid % W_PER_XROW) * PER_W, PER_W)], idx_v)

    def start_gather(g, s):
        pltpu.async_copy(table_hbm.at[idx_v.at[pl.ds(g * C, C)]],
                         buf.at[s], gsems[s])

    def wait_gather(g, s):
        pltpu.make_async_copy(table_hbm.at[idx_v.at[pl.ds(g * C, C)]],
                              buf.at[s], gsems[s]).wait()

    def start_out(g, s):
        pltpu.async_copy(buf.at[s], out_hbm.at[pl.ds(row0 + g * C, C)],
                         osems[s])

    def wait_out(g, s):
        pltpu.make_async_copy(buf.at[s],
                              out_hbm.at[pl.ds(row0 + g * C, C)],
                              osems[s]).wait()

    def scale(s):
        @pl.loop(0, C)
        def _(r):
            for c in range(D // LANES):
                sl = pl.ds(c * LANES, LANES)
                buf[s, r, sl] = buf[s, r, sl] * SCALE

    start_gather(0, 0)
    start_gather(1, 1)
    start_gather(2, 2)

    @pl.loop(0, NCHUNK // NB)
    def _(p):
        for s_off in range(NB):
            g = p * NB + s_off
            s = s_off % NB
            wait_gather(g, s)
            scale(s)
            start_out(g, s)
            if s_off == 0:
                @pl.when(p >= 1)
                def _():
                    wait_out(g - 1, (s_off - 1) % NB)
                    start_gather(g + 2, (s_off + 2) % NB)
            elif s_off == 2:
                wait_out(g - 1, (s_off - 1) % NB)
                @pl.when(g + 2 < NCHUNK)
                def _():
                    start_gather(g + 2, (s_off + 2) % NB)
            else:
                wait_out(g - 1, (s_off - 1) % NB)
                start_gather(g + 2, (s_off + 2) % NB)

    g = NCHUNK - 1
    s = g % NB
    wait_gather(g, s)
    scale(s)
    start_out(g, s)

    for g in range(NCHUNK - 2, NCHUNK):
        wait_out(g, g % NB)


def kernel(x, table):
    call = pl.kernel(
        _sc_body,
        out_type=jax.ShapeDtypeStruct((B, D), jnp.float32),
        mesh=plsc.VectorSubcoreMesh(
            core_axis_name="c", subcore_axis_name="s",
            num_cores=NC, num_subcores=NS),
        scratch_types=[
            pltpu.VMEM((PER_W,), jnp.int32),
            pltpu.VMEM((NB, C, D), jnp.float32),
        ] + [pltpu.SemaphoreType.DMA] * (3 * NB),
    )
    out = call(x.astype(jnp.int32), table)
    return out.reshape(x.shape + (D,))


# speedup vs baseline: 1.1544x; 1.0042x over previous
import jax
import jax.numpy as jnp
from jax import lax
from jax.experimental import pallas as pl
from jax.experimental.pallas import tpu as pltpu
from jax.experimental.pallas import tpu_sc as plsc

VOCAB = 100000
D = 1024
B = 4 * 4096
NC = 2
NS = 16
NW = NC * NS
PER_W = B // NW
C = 32
NCHUNK = PER_W // C
NB = 3
LANES = 16
SCALE = 32.0
XCOLS = 4096
W_PER_XROW = XCOLS // PER_W


def _sc_body(idx_hbm, table_hbm, out_hbm, idx_v, buf, *sems):
    gsems = sems[:NB]
    osems = sems[NB:]
    wid = lax.axis_index("s") * NC + lax.axis_index("c")
    row0 = wid * PER_W

    pltpu.sync_copy(
        idx_hbm.at[wid // W_PER_XROW,
                   pl.ds((wid % W_PER_XROW) * PER_W, PER_W)], idx_v)

    def start_gather(g, s):
        pltpu.async_copy(table_hbm.at[idx_v.at[pl.ds(g * C, C)]],
                         buf.at[s], gsems[s])

    def wait_gather(g, s):
        pltpu.make_async_copy(table_hbm.at[idx_v.at[pl.ds(g * C, C)]],
                              buf.at[s], gsems[s]).wait()

    def start_out(g, s):
        pltpu.async_copy(buf.at[s], out_hbm.at[pl.ds(row0 + g * C, C)],
                         osems[s])

    def wait_out(g, s):
        pltpu.make_async_copy(buf.at[s],
                              out_hbm.at[pl.ds(row0 + g * C, C)],
                              osems[s]).wait()

    def scale(s):
        @pl.loop(0, C)
        def _(r):
            for c in range(D // LANES):
                sl = pl.ds(c * LANES, LANES)
                buf[s, r, sl] = buf[s, r, sl] * SCALE

    start_gather(0, 0)
    start_gather(1, 1)
    start_gather(2, 2)

    @pl.loop(0, NCHUNK // NB)
    def _(p):
        for s_off in range(NB):
            g = p * NB + s_off
            s = s_off % NB
            wait_gather(g, s)
            scale(s)
            start_out(g, s)
            if s_off == 0:
                @pl.when(p >= 1)
                def _():
                    wait_out(g - 1, (s_off - 1) % NB)
                    start_gather(g + 2, (s_off + 2) % NB)
            elif s_off == 2:
                wait_out(g - 1, (s_off - 1) % NB)
                @pl.when(g + 2 < NCHUNK)
                def _():
                    start_gather(g + 2, (s_off + 2) % NB)
            else:
                wait_out(g - 1, (s_off - 1) % NB)
                start_gather(g + 2, (s_off + 2) % NB)

    g = NCHUNK - 1
    s = g % NB
    wait_gather(g, s)
    scale(s)
    start_out(g, s)

    for g in range(NCHUNK - 2, NCHUNK):
        wait_out(g, g % NB)


def kernel(x, table):
    call = pl.kernel(
        _sc_body,
        out_type=jax.ShapeDtypeStruct((B, D), jnp.float32),
        mesh=plsc.VectorSubcoreMesh(
            core_axis_name="c", subcore_axis_name="s",
            num_cores=NC, num_subcores=NS),
        scratch_types=[
            pltpu.VMEM((PER_W,), jnp.int32),
            pltpu.VMEM((NB, C, D), jnp.float32),
        ] + [pltpu.SemaphoreType.DMA] * (2 * NB),
    )
    out = call(x.astype(jnp.int32), table)
    return out.reshape(x.shape + (D,))
